# Initial kernel scaffold; baseline (speedup 1.0000x reference)
import jax
import jax.numpy as jnp
from jax.experimental import pallas as pl


def kernel(embeddings, cat_table, subcat_table):
    raise NotImplementedError("write your pallas kernel here")


# speedup vs baseline: 1.0132x; 1.0132x over previous
import functools

import jax
import jax.numpy as jnp
from jax import lax
from jax.experimental import pallas as pl
from jax.experimental.pallas import tpu as pltpu
from jax.experimental.pallas import tpu_sc as plsc

L = 16
NW = 32
CHUNK = 128
D_IN = 66
D_PASS = 64
D_TAB = 128
D_OUT = D_PASS + 2 * D_TAB


def _body(emb_hbm, cat_hbm, sub_hbm, out_hbm,
          emb_v, out_v, cat_v, sub_v, idxc_v, idxs_v, sem_c, sem_s):
    n_rows = emb_hbm.shape[0]
    rows_per_w = n_rows // NW
    n_chunks = rows_per_w // CHUNK

    wid = lax.axis_index("s") * 2 + lax.axis_index("c")
    w_base = wid * rows_per_w

    col_c = jnp.full((L,), D_IN - 2, jnp.int32)
    col_s = jnp.full((L,), D_IN - 1, jnp.int32)
    lane = lax.broadcasted_iota(jnp.int32, (L,), 0)

    def chunk_step(g, carry):
        base = w_base + g * CHUNK
        pltpu.sync_copy(emb_hbm.at[pl.ds(base, CHUNK)], emb_v)

        for i in range(CHUNK // L):
            rows = lane + i * L
            cf = plsc.load_gather(emb_v, [rows, col_c])
            sf = plsc.load_gather(emb_v, [rows, col_s])
            idxc_v[pl.ds(i * L, L)] = cf.astype(jnp.int32)
            idxs_v[pl.ds(i * L, L)] = sf.astype(jnp.int32)

        cpy_c = pltpu.make_async_copy(cat_hbm.at[idxc_v], cat_v, sem_c)
        cpy_s = pltpu.make_async_copy(sub_hbm.at[idxs_v], sub_v, sem_s)
        cpy_c.start()
        cpy_s.start()

        def pass_copy(r, c):
            for j in range(D_PASS // L):
                out_v[r, pl.ds(j * L, L)] = emb_v[r, pl.ds(j * L, L)]
            return c

        lax.fori_loop(0, CHUNK, pass_copy, 0)

        cpy_c.wait()
        cpy_s.wait()

        def tab_copy(r, c):
            for j in range(D_TAB // L):
                out_v[r, pl.ds(D_PASS + j * L, L)] = cat_v[r, pl.ds(j * L, L)]
                out_v[r, pl.ds(D_PASS + D_TAB + j * L, L)] = \
                    sub_v[r, pl.ds(j * L, L)]
            return c

        lax.fori_loop(0, CHUNK, tab_copy, 0)

        pltpu.sync_copy(out_v, out_hbm.at[pl.ds(base, CHUNK)])
        return carry

    lax.fori_loop(0, n_chunks, chunk_step, 0)


@jax.jit
def kernel(embeddings, cat_table, subcat_table):
    B, S, _ = embeddings.shape
    n_rows = B * S
    emb2d = embeddings.reshape(n_rows, D_IN)

    mesh = plsc.VectorSubcoreMesh(core_axis_name="c", subcore_axis_name="s")
    out2d = pl.kernel(
        _body,
        out_type=jax.ShapeDtypeStruct((n_rows, D_OUT), jnp.float32),
        mesh=mesh,
        compiler_params=pltpu.CompilerParams(needs_layout_passes=False),
        scratch_types=[
            pltpu.VMEM((CHUNK, D_IN), jnp.float32),
            pltpu.VMEM((CHUNK, D_OUT), jnp.float32),
            pltpu.VMEM((CHUNK, D_TAB), jnp.float32),
            pltpu.VMEM((CHUNK, D_TAB), jnp.float32),
            pltpu.VMEM((CHUNK,), jnp.int32),
            pltpu.VMEM((CHUNK,), jnp.int32),
            pltpu.SemaphoreType.DMA,
            pltpu.SemaphoreType.DMA,
        ],
    )(emb2d, cat_table, subcat_table)
    return out2d.reshape(B, S, D_OUT)


# speedup vs baseline: 1.7668x; 1.7437x over previous
import functools

import jax
import jax.numpy as jnp
from jax import lax
from jax.experimental import pallas as pl
from jax.experimental.pallas import tpu as pltpu
from jax.experimental.pallas import tpu_sc as plsc

L = 16
NW = 32
CHUNK = 64
D_IN = 66
D_PASS = 64
D_TAB = 128
D_CMB = 2 * D_TAB
D_OUT = D_PASS + D_CMB
N_SUB = 134
UNROLL = 8


def _body(emb_hbm, tab_hbm, out_hbm,
          e0, e1, t0, t1, o0, o1, i0, i1,
          se0, se1, sg0, sg1, so0, so1):
    n_rows = emb_hbm.shape[0] // D_IN
    rows_per_w = n_rows // NW
    n_chunks = rows_per_w // CHUNK

    wid = lax.axis_index("s") * 2 + lax.axis_index("c")
    w_base = wid * rows_per_w

    bufs = ((e0, t0, o0, i0, se0, sg0, so0),
            (e1, t1, o1, i1, se1, sg1, so1))
    lane = lax.broadcasted_iota(jnp.int32, (L,), 0)

    def in_copy(g, b):
        start = pl.multiple_of((w_base + g * CHUNK) * D_IN, 8)
        return pltpu.make_async_copy(
            emb_hbm.at[pl.ds(start, CHUNK * D_IN)], bufs[b][0], bufs[b][4])

    def out_copy(g, b):
        start = pl.multiple_of(w_base + g * CHUNK, 8)
        return pltpu.make_async_copy(
            bufs[b][2], out_hbm.at[pl.ds(start, CHUNK)], bufs[b][6])

    in_copy(0, 0).start()
    in_copy(1, 1).start()

    def step_pair(h, carry):
        for b in (0, 1):
            g = 2 * h + b
            e, t, o, iv, se, sg, so = bufs[b]

            in_copy(g, b).wait()

            for i in range(CHUNK // L):
                addr = (lane + i * L) * D_IN + (D_IN - 2)
                cf = plsc.load_gather(e, [addr])
                sf = plsc.load_gather(e, [addr + 1])
                iv[pl.ds(i * L, L)] = (cf.astype(jnp.int32) * N_SUB
                                       + sf.astype(jnp.int32))

            gather = pltpu.make_async_copy(tab_hbm.at[iv], t, sg)
            gather.start()

            @pl.when(g >= 2)
            def _():
                out_copy(g - 2, b).wait()

            def pass_copy(q, c):
                r0 = q * UNROLL
                for dr in range(UNROLL):
                    r = r0 + dr
                    for j in range(D_PASS // L):
                        o[r, pl.ds(j * L, L)] = e[pl.ds(r * D_IN + j * L, L)]
                return c

            lax.fori_loop(0, CHUNK // UNROLL, pass_copy, 0)

            @pl.when(g + 2 < n_chunks)
            def _():
                in_copy(g + 2, b).start()

            gather.wait()

            def tab_copy(q, c):
                r0 = q * UNROLL
                for dr in range(UNROLL):
                    r = r0 + dr
                    for j in range(D_CMB // L):
                        o[r, pl.ds(D_PASS + j * L, L)] = t[r, pl.ds(j * L, L)]
                return c

            lax.fori_loop(0, CHUNK // UNROLL, tab_copy, 0)

            out_copy(g, b).start()
        return carry

    lax.fori_loop(0, n_chunks // 2, step_pair, 0)
    out_copy(n_chunks - 2, 0).wait()
    out_copy(n_chunks - 1, 1).wait()


@jax.jit
def kernel(embeddings, cat_table, subcat_table):
    B, S, _ = embeddings.shape
    n_rows = B * S

    n_cat = cat_table.shape[0]
    n_sub = subcat_table.shape[0]
    tab = jnp.concatenate([
        jnp.broadcast_to(cat_table[:, None, :], (n_cat, n_sub, D_TAB)),
        jnp.broadcast_to(subcat_table[None, :, :], (n_cat, n_sub, D_TAB)),
    ], axis=-1).reshape(n_cat * n_sub, D_CMB)

    emb1d = embeddings.reshape(n_rows * D_IN)

    mesh = plsc.VectorSubcoreMesh(core_axis_name="c", subcore_axis_name="s")
    out2d = pl.kernel(
        _body,
        out_type=jax.ShapeDtypeStruct((n_rows, D_OUT), jnp.float32),
        mesh=mesh,
        compiler_params=pltpu.CompilerParams(needs_layout_passes=False),
        scratch_types=[
            pltpu.VMEM((CHUNK * D_IN,), jnp.float32),
            pltpu.VMEM((CHUNK * D_IN,), jnp.float32),
            pltpu.VMEM((CHUNK, D_CMB), jnp.float32),
            pltpu.VMEM((CHUNK, D_CMB), jnp.float32),
            pltpu.VMEM((CHUNK, D_OUT), jnp.float32),
            pltpu.VMEM((CHUNK, D_OUT), jnp.float32),
            pltpu.VMEM((CHUNK,), jnp.int32),
            pltpu.VMEM((CHUNK,), jnp.int32),
            pltpu.SemaphoreType.DMA,
            pltpu.SemaphoreType.DMA,
            pltpu.SemaphoreType.DMA,
            pltpu.SemaphoreType.DMA,
            pltpu.SemaphoreType.DMA,
            pltpu.SemaphoreType.DMA,
        ],
    )(emb1d, tab)
    return out2d.reshape(B, S, D_OUT)


# speedup vs baseline: 1.8786x; 1.0633x over previous
import functools

import jax
import jax.numpy as jnp
from jax import lax
from jax.experimental import pallas as pl
from jax.experimental.pallas import tpu as pltpu
from jax.experimental.pallas import tpu_sc as plsc

L = 16
NW = 32
CHUNK = 80
D_IN = 66
D_PASS = 64
D_TAB = 128
D_OUT = D_PASS + 2 * D_TAB
UNROLL = 8


def _body(emb_hbm, cat_hbm, sub_hbm, out_hbm,
          e0, e1, o0, o1, cat_v, sub_v, ids_s,
          se0, se1, so0, so1, sem_t):
    n_rows = emb_hbm.shape[0] // D_IN
    rows_per_w = n_rows // NW
    n_chunks = rows_per_w // CHUNK

    wid = lax.axis_index("s") * 2 + lax.axis_index("c")
    w_base = wid * rows_per_w

    bufs = ((e0, o0, se0, so0), (e1, o1, se1, so1))
    lane = lax.broadcasted_iota(jnp.int32, (L,), 0)

    pltpu.make_async_copy(cat_hbm, cat_v, sem_t).start()
    pltpu.make_async_copy(sub_hbm, sub_v, sem_t).start()
    pltpu.make_async_copy(cat_hbm, cat_v, sem_t).wait()
    pltpu.make_async_copy(sub_hbm, sub_v, sem_t).wait()

    def in_copy(g, b):
        start = pl.multiple_of((w_base + g * CHUNK) * D_IN, 8)
        return pltpu.make_async_copy(
            emb_hbm.at[pl.ds(start, CHUNK * D_IN)], bufs[b][0], bufs[b][2])

    def out_copy(g, b):
        start = pl.multiple_of(w_base + g * CHUNK, 8)
        return pltpu.make_async_copy(
            bufs[b][1], out_hbm.at[pl.ds(start, CHUNK)], bufs[b][3])

    in_copy(0, 0).start()
    in_copy(1, 1).start()

    def step_pair(h, carry):
        for b in (0, 1):
            g = 2 * h + b
            e, o, se, so = bufs[b]

            in_copy(g, b).wait()

            for i in range(CHUNK // L):
                addr = (lane + i * L) * D_IN + (D_IN - 2)
                cf = plsc.load_gather(e, [addr]).astype(jnp.int32)
                sf = plsc.load_gather(e, [addr + 1]).astype(jnp.int32)
                for k in range(L):
                    ids_s[2 * (i * L + k)] = cf[k]
                    ids_s[2 * (i * L + k) + 1] = sf[k]

            @pl.when(g >= 2)
            def _():
                out_copy(g - 2, b).wait()

            def asm(q, c):
                r0 = q * UNROLL
                for dr in range(UNROLL):
                    r = r0 + dr
                    ic = ids_s[2 * r]
                    isub = ids_s[2 * r + 1]
                    for j in range(D_PASS // L):
                        o[r, pl.ds(j * L, L)] = e[pl.ds(r * D_IN + j * L, L)]
                    for j in range(D_TAB // L):
                        o[r, pl.ds(D_PASS + j * L, L)] = \
                            cat_v[ic, pl.ds(j * L, L)]
                        o[r, pl.ds(D_PASS + D_TAB + j * L, L)] = \
                            sub_v[isub, pl.ds(j * L, L)]
                return c

            lax.fori_loop(0, CHUNK // UNROLL, asm, 0)

            @pl.when(g + 2 < n_chunks)
            def _():
                in_copy(g + 2, b).start()

            out_copy(g, b).start()
        return carry

    lax.fori_loop(0, n_chunks // 2, step_pair, 0)
    out_copy(n_chunks - 2, 0).wait()
    out_copy(n_chunks - 1, 1).wait()


@jax.jit
def kernel(embeddings, cat_table, subcat_table):
    B, S, _ = embeddings.shape
    n_rows = B * S
    emb1d = embeddings.reshape(n_rows * D_IN)

    mesh = plsc.VectorSubcoreMesh(core_axis_name="c", subcore_axis_name="s")
    out2d = pl.kernel(
        _body,
        out_type=jax.ShapeDtypeStruct((n_rows, D_OUT), jnp.float32),
        mesh=mesh,
        compiler_params=pltpu.CompilerParams(needs_layout_passes=False),
        scratch_types=[
            pltpu.VMEM((CHUNK * D_IN,), jnp.float32),
            pltpu.VMEM((CHUNK * D_IN,), jnp.float32),
            pltpu.VMEM((CHUNK, D_OUT), jnp.float32),
            pltpu.VMEM((CHUNK, D_OUT), jnp.float32),
            pltpu.VMEM((15, D_TAB), jnp.float32),
            pltpu.VMEM((134, D_TAB), jnp.float32),
            pltpu.SMEM((2 * CHUNK,), jnp.int32),
            pltpu.SemaphoreType.DMA,
            pltpu.SemaphoreType.DMA,
            pltpu.SemaphoreType.DMA,
            pltpu.SemaphoreType.DMA,
            pltpu.SemaphoreType.DMA,
        ],
    )(emb1d, cat_table, subcat_table)
    return out2d.reshape(B, S, D_OUT)


# speedup vs baseline: 2.5651x; 1.3655x over previous
import functools

import jax
import jax.numpy as jnp
from jax import lax
from jax.experimental import pallas as pl
from jax.experimental.pallas import tpu as pltpu
from jax.experimental.pallas import tpu_sc as plsc

L = 16
NW = 32
D_IN = 66
D_PASS = 64
D_TAB = 128
D_OUT = D_PASS + 2 * D_TAB
SEQ = 50
UNROLL = 5


def _body(emb_hbm, cat_hbm, sub_hbm, out_hbm,
          e0, e1, o0, o1, cat_v, sub_v, ids_s,
          se0, se1, so0, so1, sem_t):
    n_batch = emb_hbm.shape[0]
    per_w = n_batch // NW

    wid = lax.axis_index("s") * 2 + lax.axis_index("c")
    w_base = wid * per_w

    bufs = ((e0, o0, se0, so0), (e1, o1, se1, so1))
    lane = lax.broadcasted_iota(jnp.int32, (L,), 0)

    pltpu.make_async_copy(cat_hbm, cat_v, sem_t).start()
    pltpu.make_async_copy(sub_hbm, sub_v, sem_t).start()
    pltpu.make_async_copy(cat_hbm, cat_v, sem_t).wait()
    pltpu.make_async_copy(sub_hbm, sub_v, sem_t).wait()

    def in_copy(g, b):
        return pltpu.make_async_copy(
            emb_hbm.at[w_base + g], bufs[b][0], bufs[b][2])

    def out_copy(g, b):
        return pltpu.make_async_copy(
            bufs[b][1], out_hbm.at[w_base + g], bufs[b][3])

    in_copy(0, 0).start()
    in_copy(1, 1).start()

    col_c = jnp.full((L,), D_IN - 2, jnp.int32)
    col_s = jnp.full((L,), D_IN - 1, jnp.int32)

    def step_pair(h, carry):
        for b in (0, 1):
            g = 2 * h + b
            e, o, se, so = bufs[b]

            in_copy(g, b).wait()

            for i in range(4):
                base = i * L if i < 3 else SEQ - L
                rows = lane + base
                cf = plsc.load_gather(e, [rows, col_c]).astype(jnp.int32)
                sf = plsc.load_gather(e, [rows, col_s]).astype(jnp.int32)
                for k in range(L):
                    ids_s[2 * (base + k)] = cf[k]
                    ids_s[2 * (base + k) + 1] = sf[k]

            @pl.when(g >= 2)
            def _():
                out_copy(g - 2, b).wait()

            def asm(q, c):
                r0 = q * UNROLL
                for dr in range(UNROLL):
                    r = r0 + dr
                    ic = ids_s[2 * r]
                    isub = ids_s[2 * r + 1]
                    for j in range(D_PASS // L):
                        o[r, pl.ds(j * L, L)] = e[r, pl.ds(j * L, L)]
                    for j in range(D_TAB // L):
                        o[r, pl.ds(D_PASS + j * L, L)] = \
                            cat_v[ic, pl.ds(j * L, L)]
                        o[r, pl.ds(D_PASS + D_TAB + j * L, L)] = \
                            sub_v[isub, pl.ds(j * L, L)]
                return c

            lax.fori_loop(0, SEQ // UNROLL, asm, 0)

            @pl.when(g + 2 < per_w)
            def _():
                in_copy(g + 2, b).start()

            out_copy(g, b).start()
        return carry

    lax.fori_loop(0, per_w // 2, step_pair, 0)
    out_copy(per_w - 2, 0).wait()
    out_copy(per_w - 1, 1).wait()


@jax.jit
def kernel(embeddings, cat_table, subcat_table):
    B, S, _ = embeddings.shape

    mesh = plsc.VectorSubcoreMesh(core_axis_name="c", subcore_axis_name="s")
    return pl.kernel(
        _body,
        out_type=jax.ShapeDtypeStruct((B, S, D_OUT), jnp.float32),
        mesh=mesh,
        compiler_params=pltpu.CompilerParams(needs_layout_passes=False),
        scratch_types=[
            pltpu.VMEM((SEQ, D_IN), jnp.float32),
            pltpu.VMEM((SEQ, D_IN), jnp.float32),
            pltpu.VMEM((SEQ, D_OUT), jnp.float32),
            pltpu.VMEM((SEQ, D_OUT), jnp.float32),
            pltpu.VMEM((15, D_TAB), jnp.float32),
            pltpu.VMEM((134, D_TAB), jnp.float32),
            pltpu.SMEM((2 * SEQ,), jnp.int32),
            pltpu.SemaphoreType.DMA,
            pltpu.SemaphoreType.DMA,
            pltpu.SemaphoreType.DMA,
            pltpu.SemaphoreType.DMA,
            pltpu.SemaphoreType.DMA,
        ],
    )(embeddings, cat_table, subcat_table)
